# Initial kernel scaffold; baseline (speedup 1.0000x reference)
import functools

import jax
import jax.numpy as jnp
from jax import lax
from jax.experimental import pallas as pl
from jax.experimental.pallas import tpu as pltpu
from jax.experimental.pallas import tpu_sc as plsc

RANKS = 13
SUITS = 4
RANK_DIM = 8
SUIT_DIM = 4
OUT_DIM = 16
BATCH = 16384

NC = 2
NS = 16
LANES = 16
NW = NC * NS
BPW = BATCH // NW
CHUNK = 128
NCHUNK = BPW // CHUNK


def _fold_body(rank_ref, suit_ref, w_ref, b_ref, out_ref):
    tr = jnp.dot(rank_ref[...], w_ref[:RANK_DIM, :],
                 preferred_element_type=jnp.float32) + b_ref[...]
    ts = jnp.dot(suit_ref[...], w_ref[RANK_DIM:, :],
                 preferred_element_type=jnp.float32)
    out_ref[...] = tr[:, None, :] + ts[None, :, :]


def _fold_tables(rank_emb, suit_emb, W, b):
    t3 = pl.pallas_call(
        _fold_body,
        out_shape=jax.ShapeDtypeStruct((RANKS, SUITS, OUT_DIM), jnp.float32),
    )(rank_emb, suit_emb, W, b.reshape(1, OUT_DIM))
    return t3.reshape(RANKS * SUITS, OUT_DIM)


def _sc_body(card_hbm, table_hbm, out_hbm, card_v, idx_v, rows_v, sem):
    wid = lax.axis_index("s") * NC + lax.axis_index("c")
    base = wid * BPW
    pltpu.sync_copy(card_hbm.at[pl.ds(2 * base, 2 * BPW)], card_v)
    lane = lax.iota(jnp.int32, LANES)
    for j in range(BPW // LANES):
        pos = lane * 2 + (2 * LANES * j)
        r = plsc.load_gather(card_v, [pos])
        s = plsc.load_gather(card_v, [pos + 1])
        idx_v[j // (CHUNK // LANES),
              pl.ds((j % (CHUNK // LANES)) * LANES, LANES)] = r * SUITS + s
    copies = [
        pltpu.async_copy(table_hbm.at[idx_v.at[j]],
                         rows_v.at[pl.ds(j * CHUNK, CHUNK)], sem)
        for j in range(NCHUNK)
    ]
    for c in copies:
        c.wait()
    pltpu.sync_copy(rows_v, out_hbm.at[pl.ds(base, BPW)])


_sc_lookup = functools.partial(
    pl.kernel,
    mesh=plsc.VectorSubcoreMesh(core_axis_name="c", subcore_axis_name="s"),
    out_type=jax.ShapeDtypeStruct((BATCH, OUT_DIM), jnp.float32),
    scratch_types=[
        pltpu.VMEM((2 * BPW,), jnp.int32),
        pltpu.VMEM((NCHUNK, CHUNK), jnp.int32),
        pltpu.VMEM((BPW, OUT_DIM), jnp.float32),
        pltpu.SemaphoreType.DMA,
    ],
)(_sc_body)


def kernel(card_tensor, rank_emb, suit_emb, W, b):
    table = _fold_tables(rank_emb, suit_emb, W, b)
    card_flat = card_tensor.astype(jnp.int32).reshape(-1)
    return _sc_lookup(card_flat, table)


# speedup vs baseline: 1.3347x; 1.3347x over previous
import functools

import jax
import jax.numpy as jnp
from jax import lax
from jax.experimental import pallas as pl
from jax.experimental.pallas import tpu as pltpu
from jax.experimental.pallas import tpu_sc as plsc

RANKS = 13
SUITS = 4
RANK_DIM = 8
SUIT_DIM = 4
OUT_DIM = 16
BATCH = 16384

NC = 2
NS = 16
LANES = 16
NW = NC * NS
BPW = BATCH // NW
CHUNK = 128
NCHUNK = BPW // CHUNK


def _fold_body(rank_ref, suit_ref, w_ref, b_ref, out_ref):
    tr = jnp.dot(rank_ref[...], w_ref[:RANK_DIM, :],
                 preferred_element_type=jnp.float32) + b_ref[...]
    ts = jnp.dot(suit_ref[...], w_ref[RANK_DIM:, :],
                 preferred_element_type=jnp.float32)
    out_ref[...] = tr[:, None, :] + ts[None, :, :]


def _fold_tables(rank_emb, suit_emb, W, b):
    t3 = pl.pallas_call(
        _fold_body,
        out_shape=jax.ShapeDtypeStruct((RANKS, SUITS, OUT_DIM), jnp.float32),
    )(rank_emb, suit_emb, W, b.reshape(1, OUT_DIM))
    return t3.reshape(RANKS * SUITS, OUT_DIM)


def _sc_body(card_hbm, table_hbm, out_hbm, card_v, idx_v, rows_v, sem):
    wid = lax.axis_index("s") * NC + lax.axis_index("c")
    base = wid * BPW
    pltpu.sync_copy(card_hbm.at[pl.ds(2 * base, 2 * BPW)], card_v)
    lane = lax.iota(jnp.int32, LANES)
    for j in range(BPW // LANES):
        pos = lane * 2 + (2 * LANES * j)
        r = plsc.load_gather(card_v, [pos])
        s = plsc.load_gather(card_v, [pos + 1])
        idx_v[j // (CHUNK // LANES),
              pl.ds((j % (CHUNK // LANES)) * LANES, LANES)] = r * SUITS + s
    copies = [
        pltpu.async_copy(table_hbm.at[idx_v.at[j]],
                         rows_v.at[pl.ds(j * CHUNK, CHUNK)], sem)
        for j in range(NCHUNK)
    ]
    for c in copies:
        c.wait()
    pltpu.sync_copy(rows_v, out_hbm.at[pl.ds(base, BPW)])


@functools.lru_cache(maxsize=1)
def _sc_lookup():
    return pl.kernel(
        _sc_body,
        mesh=plsc.VectorSubcoreMesh(core_axis_name="c", subcore_axis_name="s"),
        out_type=jax.ShapeDtypeStruct((BATCH, OUT_DIM), jnp.float32),
        scratch_types=[
            pltpu.VMEM((2 * BPW,), jnp.int32),
            pltpu.VMEM((NCHUNK, CHUNK), jnp.int32),
            pltpu.VMEM((BPW, OUT_DIM), jnp.float32),
            pltpu.SemaphoreType.DMA,
        ],
        compiler_params=pltpu.CompilerParams(
            needs_layout_passes=False, use_tc_tiling_on_sc=False),
    )


def kernel(card_tensor, rank_emb, suit_emb, W, b):
    table = _fold_tables(rank_emb, suit_emb, W, b)
    card_flat = card_tensor.astype(jnp.int32).reshape(-1)
    return _sc_lookup()(card_flat, table)


# speedup vs baseline: 1.3472x; 1.0094x over previous
import functools

import jax
import jax.numpy as jnp
from jax import lax
from jax.experimental import pallas as pl
from jax.experimental.pallas import tpu as pltpu
from jax.experimental.pallas import tpu_sc as plsc

RANKS = 13
SUITS = 4
RANK_DIM = 8
SUIT_DIM = 4
OUT_DIM = 16
BATCH = 16384

NC = 1
NS = 16
LANES = 16
NW = NC * NS
BPW = BATCH // NW
CHUNK = 128
NCHUNK = BPW // CHUNK


def _fold_body(rank_ref, suit_ref, w_ref, b_ref, out_ref):
    tr = jnp.dot(rank_ref[...], w_ref[:RANK_DIM, :],
                 preferred_element_type=jnp.float32) + b_ref[...]
    ts = jnp.dot(suit_ref[...], w_ref[RANK_DIM:, :],
                 preferred_element_type=jnp.float32)
    out_ref[...] = tr[:, None, :] + ts[None, :, :]


def _fold_tables(rank_emb, suit_emb, W, b):
    t3 = pl.pallas_call(
        _fold_body,
        out_shape=jax.ShapeDtypeStruct((RANKS, SUITS, OUT_DIM), jnp.float32),
    )(rank_emb, suit_emb, W, b.reshape(1, OUT_DIM))
    return t3.reshape(RANKS * SUITS, OUT_DIM)


def _sc_body(card_hbm, table_hbm, out_hbm, card_v, idx_v, rows_v, sem):
    wid = lax.axis_index("s") * NC + lax.axis_index("c")
    base = wid * BPW
    pltpu.sync_copy(card_hbm.at[pl.ds(2 * base, 2 * BPW)], card_v)
    lane = lax.iota(jnp.int32, LANES)
    for j in range(BPW // LANES):
        pos = lane * 2 + (2 * LANES * j)
        r = plsc.load_gather(card_v, [pos])
        s = plsc.load_gather(card_v, [pos + 1])
        idx_v[j // (CHUNK // LANES),
              pl.ds((j % (CHUNK // LANES)) * LANES, LANES)] = r * SUITS + s
    copies = [
        pltpu.async_copy(table_hbm.at[idx_v.at[j]],
                         rows_v.at[pl.ds(j * CHUNK, CHUNK)], sem)
        for j in range(NCHUNK)
    ]
    for c in copies:
        c.wait()
    pltpu.sync_copy(rows_v, out_hbm.at[pl.ds(base, BPW)])


@functools.lru_cache(maxsize=1)
def _sc_lookup():
    return pl.kernel(
        _sc_body,
        mesh=plsc.VectorSubcoreMesh(core_axis_name="c", subcore_axis_name="s",
                                    num_cores=NC),
        out_type=jax.ShapeDtypeStruct((BATCH, OUT_DIM), jnp.float32),
        scratch_types=[
            pltpu.VMEM((2 * BPW,), jnp.int32),
            pltpu.VMEM((NCHUNK, CHUNK), jnp.int32),
            pltpu.VMEM((BPW, OUT_DIM), jnp.float32),
            pltpu.SemaphoreType.DMA,
        ],
        compiler_params=pltpu.CompilerParams(
            needs_layout_passes=False, use_tc_tiling_on_sc=False),
    )


def kernel(card_tensor, rank_emb, suit_emb, W, b):
    table = _fold_tables(rank_emb, suit_emb, W, b)
    card_flat = card_tensor.astype(jnp.int32).reshape(-1)
    return _sc_lookup()(card_flat, table)


# speedup vs baseline: 1.9597x; 1.4546x over previous
import functools

import jax
import jax.numpy as jnp
from jax import lax
from jax.experimental import pallas as pl
from jax.experimental.pallas import tpu as pltpu
from jax.experimental.pallas import tpu_sc as plsc

RANKS = 13
SUITS = 4
RANK_DIM = 8
SUIT_DIM = 4
OUT_DIM = 16
BATCH = 16384

NC = 1
NS = 16
LANES = 16
NW = NC * NS
BPW = BATCH // NW
CHUNK = 128
NCHUNK = BPW // CHUNK


def _fold_body(rank_ref, suit_ref, w_ref, b_ref, out_ref):
    tr = jnp.dot(rank_ref[...], w_ref[:RANK_DIM, :],
                 preferred_element_type=jnp.float32) + b_ref[...]
    ts = jnp.dot(suit_ref[...], w_ref[RANK_DIM:, :],
                 preferred_element_type=jnp.float32)
    out_ref[...] = tr[:, None, :] + ts[None, :, :]


def _fold_tables(rank_emb, suit_emb, W, b):
    t3 = pl.pallas_call(
        _fold_body,
        out_shape=jax.ShapeDtypeStruct((RANKS, SUITS, OUT_DIM), jnp.float32),
    )(rank_emb, suit_emb, W, b.reshape(1, OUT_DIM))
    return t3.reshape(RANKS * SUITS, OUT_DIM)


def _sc_body(card_hbm, table_hbm, out_hbm, card_v, table_v, rows_v, sem):
    wid = lax.axis_index("s") * NC + lax.axis_index("c")
    base = wid * BPW
    ccopy = pltpu.async_copy(card_hbm.at[pl.ds(2 * base, 2 * BPW)], card_v, sem)
    pltpu.sync_copy(table_hbm, table_v)
    ccopy.wait()
    lane = lax.iota(jnp.int32, LANES)
    lane16 = lane * OUT_DIM
    for j in range(BPW // LANES):
        pos = lane * 2 + (2 * LANES * j)
        r = plsc.load_gather(card_v, [pos])
        s = plsc.load_gather(card_v, [pos + 1])
        tbase = (r * SUITS + s) * OUT_DIM
        obase = lane16 + j * LANES * OUT_DIM
        for c in range(OUT_DIM):
            vals = plsc.load_gather(table_v, [tbase + c])
            plsc.store_scatter(rows_v, [obase + c], vals)
    pltpu.sync_copy(rows_v, out_hbm.at[pl.ds(base * OUT_DIM, BPW * OUT_DIM)])


@functools.lru_cache(maxsize=1)
def _sc_lookup():
    return pl.kernel(
        _sc_body,
        mesh=plsc.VectorSubcoreMesh(core_axis_name="c", subcore_axis_name="s",
                                    num_cores=NC),
        out_type=jax.ShapeDtypeStruct((BATCH * OUT_DIM,), jnp.float32),
        scratch_types=[
            pltpu.VMEM((2 * BPW,), jnp.int32),
            pltpu.VMEM((RANKS * SUITS * OUT_DIM,), jnp.float32),
            pltpu.VMEM((BPW * OUT_DIM,), jnp.float32),
            pltpu.SemaphoreType.DMA,
        ],
        compiler_params=pltpu.CompilerParams(
            needs_layout_passes=False, use_tc_tiling_on_sc=False),
    )


def kernel(card_tensor, rank_emb, suit_emb, W, b):
    table = _fold_tables(rank_emb, suit_emb, W, b)
    card_flat = card_tensor.astype(jnp.int32).reshape(-1)
    out_flat = _sc_lookup()(card_flat, table.reshape(-1))
    return out_flat.reshape(BATCH, OUT_DIM)
